# baseline (speedup 1.0000x reference)
import functools

import jax
import jax.numpy as jnp
from jax import lax
from jax.experimental import pallas as pl
from jax.experimental.pallas import tpu as pltpu
from jax.experimental.pallas import tpu_sc as plsc

_N, _E, _D, _H, _O, _G = 10000, 320000, 128, 32, 2, 64
_NP = _N // 4
_NW = 32
_CHUNK = 125
_KCH = 80
_NPAD = 10112
_PPAD = _NPAD // 4
_RPT = _NPAD // 16
_NBUF = 8
_DEPTH = 4


def _proj_body(x_ref, w_ref, o_ref):
    o_ref[...] = jnp.dot(x_ref[...], w_ref[...],
                         preferred_element_type=jnp.float32)


def _project(x4, w4):
    return pl.pallas_call(
        _proj_body,
        out_shape=jax.ShapeDtypeStruct((_NP, 128), jnp.float32),
    )(x4, w4)


def _mid_body(y_ref, p0_ref, p1_ref, ba_ref, wb_ref, bb_ref, wn_ref, o_ref):
    m = jnp.maximum(y_ref[...] + p0_ref[...] + p1_ref[...] + ba_ref[...], 0.0)
    h = jnp.maximum(
        jnp.dot(m, wb_ref[...], preferred_element_type=jnp.float32)
        + bb_ref[...], 0.0)
    o_ref[...] = jnp.dot(h, wn_ref[...], preferred_element_type=jnp.float32)


def _mid(y, p0, p1, ba4, wb4, bb4, wn4):
    return pl.pallas_call(
        _mid_body,
        grid=(1,),
        in_specs=[
            pl.BlockSpec((_NP, 128), lambda i: (0, 0)),
            pl.BlockSpec((_NP, 128), lambda i: (0, 0)),
            pl.BlockSpec((_NP, 128), lambda i: (0, 0)),
            pl.BlockSpec((1, 128), lambda i: (0, 0)),
            pl.BlockSpec((128, 128), lambda i: (0, 0)),
            pl.BlockSpec((1, 128), lambda i: (0, 0)),
            pl.BlockSpec((128, 128), lambda i: (0, 0)),
        ],
        out_specs=pl.BlockSpec((_NP, 128), lambda i: (0, 0)),
        out_shape=jax.ShapeDtypeStruct((_NP, 128), jnp.float32),
    )(y, p0, p1, ba4, wb4, bb4, wn4)


def _final_body(y_ref, p0_ref, p1_ref, ba_ref, wb_ref, bb_ref, b4_ref,
                wc1_ref, bc1_ref, wc2_ref, bc2_ref, o_ref):
    m = jnp.maximum(y_ref[...] + p0_ref[...] + p1_ref[...] + ba_ref[...], 0.0)
    h = jnp.maximum(
        jnp.dot(m, wb_ref[...], preferred_element_type=jnp.float32)
        + bb_ref[...], 0.0)
    lanes = lax.broadcasted_iota(jnp.int32, (_NP, 128), 1) // _H
    sums128 = jnp.zeros((_G, 128), jnp.float32)
    cnt = jnp.zeros((_G, 1), jnp.float32)
    for k in range(4):
        bk = b4_ref[k, :]
        ohtk = (bk[None, :] == lax.broadcasted_iota(jnp.int32, (_G, _NP), 0)
                ).astype(jnp.float32)
        hk = jnp.where(lanes == k, h, 0.0)
        sums128 += jnp.dot(ohtk, hk, preferred_element_type=jnp.float32)
        cnt += jnp.sum(ohtk, axis=1)[:, None]
    sums = (sums128[:, 0:32] + sums128[:, 32:64]
            + sums128[:, 64:96] + sums128[:, 96:128])
    pooled = sums / jnp.maximum(cnt, 1.0)
    z1 = jnp.maximum(
        jnp.dot(pooled, wc1_ref[...], preferred_element_type=jnp.float32)
        + bc1_ref[...], 0.0)
    z = jnp.dot(z1, wc2_ref[...],
                preferred_element_type=jnp.float32) + bc2_ref[...]
    mx = jnp.max(z, axis=1, keepdims=True)
    e = jnp.exp(z - mx)
    o_ref[...] = z - mx - jnp.log(jnp.sum(e, axis=1, keepdims=True))


def _final(y, p0, p1, ba4, wb4, bb4, b4, wc1, bc1, wc2, bc2):
    return pl.pallas_call(
        _final_body,
        grid=(1,),
        in_specs=[
            pl.BlockSpec((_NP, 128), lambda i: (0, 0)),
            pl.BlockSpec((_NP, 128), lambda i: (0, 0)),
            pl.BlockSpec((_NP, 128), lambda i: (0, 0)),
            pl.BlockSpec((1, 128), lambda i: (0, 0)),
            pl.BlockSpec((128, 128), lambda i: (0, 0)),
            pl.BlockSpec((1, 128), lambda i: (0, 0)),
            pl.BlockSpec((4, _NP), lambda i: (0, 0)),
            pl.BlockSpec((_H, _H), lambda i: (0, 0)),
            pl.BlockSpec((1, _H), lambda i: (0, 0)),
            pl.BlockSpec((_H, _O), lambda i: (0, 0)),
            pl.BlockSpec((1, _O), lambda i: (0, 0)),
        ],
        out_specs=pl.BlockSpec((_G, _O), lambda i: (0, 0)),
        out_shape=jax.ShapeDtypeStruct((_G, _O), jnp.float32),
    )(y, p0, p1, ba4, wb4, bb4, b4,
      wc1, bc1.reshape(1, _H), wc2, bc2.reshape(1, _O))


_sc_mesh = plsc.VectorSubcoreMesh(core_axis_name="c", subcore_axis_name="s")


@functools.partial(
    pl.kernel,
    mesh=_sc_mesh,
    compiler_params=pltpu.CompilerParams(use_tc_tiling_on_sc=False),
    out_type=(jax.ShapeDtypeStruct((_NPAD, _H), jnp.float32),
              jax.ShapeDtypeStruct((_NPAD, _H), jnp.float32)),
    scratch_types=[
        pltpu.VMEM((_KCH, _CHUNK), jnp.int32),
        pltpu.VMEM((_KCH, _CHUNK), jnp.int32),
        pltpu.VMEM((_NBUF, _CHUNK, _H), jnp.float32),
        pltpu.VMEM_SHARED((_NPAD, _H), jnp.float32),
        pltpu.VMEM_SHARED((_N, _H), jnp.float32),
        pltpu.SemaphoreType.DMA,
        pltpu.SemaphoreType.DMA,
    ],
)
def _sc_agg(y_hbm, zeros_hbm, ei_hbm, out0_hbm, out1_hbm,
            sidx, didx, rows, acc, y_sh, gsem, ssem):
    cid = lax.axis_index("c")
    sid = lax.axis_index("s")
    w = sid * 2 + cid
    r0 = sid * _RPT
    ry = _N // 16

    pltpu.sync_copy(zeros_hbm.at[pl.ds(r0, _RPT)], acc.at[pl.ds(r0, _RPT)])
    pltpu.sync_copy(y_hbm.at[pl.ds(sid * ry, ry)], y_sh.at[pl.ds(sid * ry, ry)])
    pltpu.sync_copy(ei_hbm.at[pl.ds(w * _KCH, _KCH)], sidx)
    pltpu.sync_copy(ei_hbm.at[pl.ds((_NW + w) * _KCH, _KCH)], didx)
    plsc.subcore_barrier()

    for b in range(_DEPTH):
        pltpu.async_copy(y_sh.at[sidx.at[b]], rows.at[b], gsem)

    def body(it, carry):
        base = it * _NBUF
        for b in range(_NBUF):
            j = base + b
            pltpu.make_async_copy(y_sh.at[sidx.at[j]], rows.at[b],
                                  gsem).wait()
            pltpu.async_copy(rows.at[b], acc.at[didx.at[j]], ssem, add=True)

            @pl.when(j >= _DEPTH)
            def _():
                pltpu.make_async_copy(rows.at[b], acc.at[didx.at[j]],
                                      ssem).wait()

            nj = j + _DEPTH
            nb = (b + _DEPTH) % _NBUF

            @pl.when(nj < _KCH)
            def _():
                pltpu.async_copy(y_sh.at[sidx.at[nj]], rows.at[nb], gsem)
        return carry

    lax.fori_loop(0, _KCH // _NBUF, body, 0)
    for _ in range(_DEPTH):
        pltpu.make_async_copy(rows.at[0], acc.at[didx.at[0]], ssem).wait()
    plsc.subcore_barrier()

    @pl.when(cid == 0)
    def _():
        pltpu.sync_copy(acc.at[pl.ds(r0, _RPT)], out0_hbm.at[pl.ds(r0, _RPT)])

    @pl.when(cid == 1)
    def _():
        pltpu.sync_copy(acc.at[pl.ds(r0, _RPT)], out1_hbm.at[pl.ds(r0, _RPT)])


def _kron4(w):
    return jnp.kron(jnp.eye(4, dtype=jnp.float32), w)


def kernel(x, edge_index, batch, W0_0, b0_0, W0_1, b0_1, W1_0, b1_0,
           W1_1, b1_1, W2_0, b2_0, W2_1, b2_1, Wc1, bc1, Wc2, bc2):
    ei = edge_index.reshape(2 * _NW * _KCH, _CHUNK)
    zeros = jnp.zeros((_NPAD, _H), jnp.float32)
    x4 = x.reshape(_NP, 4 * _D)
    b4 = batch.reshape(_NP, 4).T
    w0_4 = _kron4(W0_0)
    wb0, wn0 = _kron4(W0_1), _kron4(W1_0)
    wb1, wn1 = _kron4(W1_1), _kron4(W2_0)
    wb2 = _kron4(W2_1)
    ba0, bb0 = jnp.tile(b0_0, 4)[None], jnp.tile(b0_1, 4)[None]
    ba1, bb1 = jnp.tile(b1_0, 4)[None], jnp.tile(b1_1, 4)[None]
    ba2, bb2 = jnp.tile(b2_0, 4)[None], jnp.tile(b2_1, 4)[None]

    def packed(p):
        return p.reshape(_PPAD, 128)[:_NP]

    y0 = _project(x4, w0_4)
    p0a, p0b = _sc_agg(y0.reshape(_N, _H), zeros, ei)
    y1 = _mid(y0, packed(p0a), packed(p0b), ba0, wb0, bb0, wn0)
    p1a, p1b = _sc_agg(y1.reshape(_N, _H), zeros, ei)
    y2 = _mid(y1, packed(p1a), packed(p1b), ba1, wb1, bb1, wn1)
    p2a, p2b = _sc_agg(y2.reshape(_N, _H), zeros, ei)
    return _final(y2, packed(p2a), packed(p2b), ba2, wb2, bb2, b4,
                  Wc1, bc1, Wc2, bc2)


# speedup vs baseline: 1.4055x; 1.4055x over previous
import functools

import jax
import jax.numpy as jnp
from jax import lax
from jax.experimental import pallas as pl
from jax.experimental.pallas import tpu as pltpu
from jax.experimental.pallas import tpu_sc as plsc

_N, _E, _D, _H, _O, _G = 10000, 320000, 128, 32, 2, 64
_NP = _N // 4
_NW = 32
_CHUNK = 125
_KCH = 80
_RPT = _N // 16
_NBUF = 8
_DEPTH = 4


def _proj_body(x_ref, w_ref, o_ref):
    o_ref[...] = jnp.dot(x_ref[...], w_ref[...],
                         preferred_element_type=jnp.float32)


def _project(x4, w4):
    return pl.pallas_call(
        _proj_body,
        out_shape=jax.ShapeDtypeStruct((_NP, 128), jnp.float32),
    )(x4, w4)


def _mid_body(y_ref, p0_ref, p1_ref, ba_ref, wb_ref, bb_ref, wn_ref, o_ref):
    m = jnp.maximum(y_ref[...] + p0_ref[...] + p1_ref[...] + ba_ref[...], 0.0)
    h = jnp.maximum(
        jnp.dot(m, wb_ref[...], preferred_element_type=jnp.float32)
        + bb_ref[...], 0.0)
    o_ref[...] = jnp.dot(h, wn_ref[...], preferred_element_type=jnp.float32)


def _mid(y, p0, p1, ba4, wb4, bb4, wn4):
    return pl.pallas_call(
        _mid_body,
        grid=(1,),
        in_specs=[
            pl.BlockSpec((_NP, 128), lambda i: (0, 0)),
            pl.BlockSpec((_NP, 128), lambda i: (0, 0)),
            pl.BlockSpec((_NP, 128), lambda i: (0, 0)),
            pl.BlockSpec((1, 128), lambda i: (0, 0)),
            pl.BlockSpec((128, 128), lambda i: (0, 0)),
            pl.BlockSpec((1, 128), lambda i: (0, 0)),
            pl.BlockSpec((128, 128), lambda i: (0, 0)),
        ],
        out_specs=pl.BlockSpec((_NP, 128), lambda i: (0, 0)),
        out_shape=jax.ShapeDtypeStruct((_NP, 128), jnp.float32),
    )(y, p0, p1, ba4, wb4, bb4, wn4)


def _final_body(y_ref, p0_ref, p1_ref, ba_ref, wb_ref, bb_ref, b4_ref,
                wc1_ref, bc1_ref, wc2_ref, bc2_ref, o_ref):
    m = jnp.maximum(y_ref[...] + p0_ref[...] + p1_ref[...] + ba_ref[...], 0.0)
    h = jnp.maximum(
        jnp.dot(m, wb_ref[...], preferred_element_type=jnp.float32)
        + bb_ref[...], 0.0)
    lanes = lax.broadcasted_iota(jnp.int32, (_NP, 128), 1) // _H
    sums128 = jnp.zeros((_G, 128), jnp.float32)
    cnt = jnp.zeros((_G, 1), jnp.float32)
    for k in range(4):
        bk = b4_ref[k, :]
        ohtk = (bk[None, :] == lax.broadcasted_iota(jnp.int32, (_G, _NP), 0)
                ).astype(jnp.float32)
        hk = jnp.where(lanes == k, h, 0.0)
        sums128 += jnp.dot(ohtk, hk, preferred_element_type=jnp.float32)
        cnt += jnp.sum(ohtk, axis=1)[:, None]
    sums = (sums128[:, 0:32] + sums128[:, 32:64]
            + sums128[:, 64:96] + sums128[:, 96:128])
    pooled = sums / jnp.maximum(cnt, 1.0)
    z1 = jnp.maximum(
        jnp.dot(pooled, wc1_ref[...], preferred_element_type=jnp.float32)
        + bc1_ref[...], 0.0)
    z = jnp.dot(z1, wc2_ref[...],
                preferred_element_type=jnp.float32) + bc2_ref[...]
    mx = jnp.max(z, axis=1, keepdims=True)
    e = jnp.exp(z - mx)
    o_ref[...] = z - mx - jnp.log(jnp.sum(e, axis=1, keepdims=True))


def _final(y, p0, p1, ba4, wb4, bb4, b4, wc1, bc1, wc2, bc2):
    return pl.pallas_call(
        _final_body,
        grid=(1,),
        in_specs=[
            pl.BlockSpec((_NP, 128), lambda i: (0, 0)),
            pl.BlockSpec((_NP, 128), lambda i: (0, 0)),
            pl.BlockSpec((_NP, 128), lambda i: (0, 0)),
            pl.BlockSpec((1, 128), lambda i: (0, 0)),
            pl.BlockSpec((128, 128), lambda i: (0, 0)),
            pl.BlockSpec((1, 128), lambda i: (0, 0)),
            pl.BlockSpec((4, _NP), lambda i: (0, 0)),
            pl.BlockSpec((_H, _H), lambda i: (0, 0)),
            pl.BlockSpec((1, _H), lambda i: (0, 0)),
            pl.BlockSpec((_H, _O), lambda i: (0, 0)),
            pl.BlockSpec((1, _O), lambda i: (0, 0)),
        ],
        out_specs=pl.BlockSpec((_G, _O), lambda i: (0, 0)),
        out_shape=jax.ShapeDtypeStruct((_G, _O), jnp.float32),
    )(y, p0, p1, ba4, wb4, bb4, b4,
      wc1, bc1.reshape(1, _H), wc2, bc2.reshape(1, _O))


_sc_mesh = plsc.VectorSubcoreMesh(core_axis_name="c", subcore_axis_name="s")


@functools.partial(
    pl.kernel,
    mesh=_sc_mesh,
    compiler_params=pltpu.CompilerParams(use_tc_tiling_on_sc=False),
    out_type=(jax.ShapeDtypeStruct((_N, _H), jnp.float32),
              jax.ShapeDtypeStruct((_N, _H), jnp.float32)),
    scratch_types=[
        pltpu.VMEM((_KCH, _CHUNK), jnp.int32),
        pltpu.VMEM((_KCH, _CHUNK), jnp.int32),
        pltpu.VMEM((_NBUF, _CHUNK, _H), jnp.float32),
        pltpu.VMEM_SHARED((_N, _H), jnp.float32),
        pltpu.VMEM_SHARED((_N, _H), jnp.float32),
        pltpu.SemaphoreType.DMA,
        pltpu.SemaphoreType.DMA,
    ],
)
def _sc_agg(y_hbm, zeros_hbm, ei_hbm, out0_hbm, out1_hbm,
            sidx, didx, rows, acc, y_sh, gsem, ssem):
    cid = lax.axis_index("c")
    sid = lax.axis_index("s")
    w = sid * 2 + cid
    r0 = sid * _RPT
    ry = _N // 16

    pltpu.sync_copy(zeros_hbm.at[pl.ds(r0, _RPT)], acc.at[pl.ds(r0, _RPT)])
    pltpu.sync_copy(y_hbm.at[pl.ds(sid * ry, ry)], y_sh.at[pl.ds(sid * ry, ry)])
    pltpu.sync_copy(ei_hbm.at[pl.ds(w * _KCH, _KCH)], sidx)
    pltpu.sync_copy(ei_hbm.at[pl.ds((_NW + w) * _KCH, _KCH)], didx)
    plsc.subcore_barrier()

    for b in range(_DEPTH):
        pltpu.async_copy(y_sh.at[sidx.at[b]], rows.at[b], gsem)

    def body(it, carry):
        base = it * _NBUF
        for b in range(_NBUF):
            j = base + b
            pltpu.make_async_copy(y_sh.at[sidx.at[j]], rows.at[b],
                                  gsem).wait()
            pltpu.async_copy(rows.at[b], acc.at[didx.at[j]], ssem, add=True)

            @pl.when(j >= _DEPTH)
            def _():
                pltpu.make_async_copy(rows.at[b], acc.at[didx.at[j]],
                                      ssem).wait()

            nj = j + _DEPTH
            nb = (b + _DEPTH) % _NBUF

            @pl.when(nj < _KCH)
            def _():
                pltpu.async_copy(y_sh.at[sidx.at[nj]], rows.at[nb], gsem)
        return carry

    lax.fori_loop(0, _KCH // _NBUF, body, 0)
    for _ in range(_DEPTH):
        pltpu.make_async_copy(rows.at[0], acc.at[didx.at[0]], ssem).wait()
    plsc.subcore_barrier()

    @pl.when(cid == 0)
    def _():
        pltpu.sync_copy(acc.at[pl.ds(r0, _RPT)], out0_hbm.at[pl.ds(r0, _RPT)])

    @pl.when(cid == 1)
    def _():
        pltpu.sync_copy(acc.at[pl.ds(r0, _RPT)], out1_hbm.at[pl.ds(r0, _RPT)])


def _kron4(w):
    return jnp.kron(jnp.eye(4, dtype=jnp.float32), w)


def kernel(x, edge_index, batch, W0_0, b0_0, W0_1, b0_1, W1_0, b1_0,
           W1_1, b1_1, W2_0, b2_0, W2_1, b2_1, Wc1, bc1, Wc2, bc2):
    ei = edge_index.reshape(2 * _NW * _KCH, _CHUNK)
    zeros = jnp.zeros((_N, _H), jnp.float32)
    x4 = x.reshape(_NP, 4 * _D)
    b4 = jnp.stack([batch[k::4] for k in range(4)], axis=0)
    w0_4 = _kron4(W0_0)
    wb0, wn0 = _kron4(W0_1), _kron4(W1_0)
    wb1, wn1 = _kron4(W1_1), _kron4(W2_0)
    wb2 = _kron4(W2_1)
    ba0, bb0 = jnp.tile(b0_0, 4)[None], jnp.tile(b0_1, 4)[None]
    ba1, bb1 = jnp.tile(b1_0, 4)[None], jnp.tile(b1_1, 4)[None]
    ba2, bb2 = jnp.tile(b2_0, 4)[None], jnp.tile(b2_1, 4)[None]

    def packed(p):
        return p.reshape(_NP, 128)

    y0 = _project(x4, w0_4)
    p0a, p0b = _sc_agg(y0.reshape(_N, _H), zeros, ei)
    y1 = _mid(y0, packed(p0a), packed(p0b), ba0, wb0, bb0, wn0)
    p1a, p1b = _sc_agg(y1.reshape(_N, _H), zeros, ei)
    y2 = _mid(y1, packed(p1a), packed(p1b), ba1, wb1, bb1, wn1)
    p2a, p2b = _sc_agg(y2.reshape(_N, _H), zeros, ei)
    return _final(y2, packed(p2a), packed(p2b), ba2, wb2, bb2, b4,
                  Wc1, bc1, Wc2, bc2)
